# baseline (speedup 1.0000x reference)
import functools

import jax
import jax.numpy as jnp
from jax import lax
from jax.experimental import pallas as pl
from jax.experimental.pallas import tpu as pltpu
from jax.experimental.pallas import tpu_sc as plsc

NC, NS, L = 2, 16, 16
NW = NC * NS

ROWS, COLS = 2048, 16384
N = ROWS * COLS
KTOT = 32 * ROWS

NPW = N // NW
CHUNK = 32768
NCHUNK = NPW // CHUNK

NB1 = 256
NB2 = 4096
NB3 = 4096

_mesh = plsc.VectorSubcoreMesh(
    core_axis_name="c", subcore_axis_name="s", num_cores=NC, num_subcores=NS
)


def _zero_i32(ref, n):
    z = jnp.zeros((16,), jnp.int32)

    def body(i, _):
        ref[pl.ds(i * 16, 16)] = z
        return 0

    lax.fori_loop(0, n // 16, body, 0)


def _select_hist(comb, nb, k_need):
    ngroups = nb // 16

    def gbody(i, carry):
        acc, g_sel, acc_at = carry
        g = ngroups - 1 - i
        v = comb[pl.ds(g * 16, 16)]
        s = jnp.sum(v)
        hit = jnp.logical_and(acc < k_need, acc + s >= k_need)
        g_sel = jnp.where(hit, g, g_sel)
        acc_at = jnp.where(hit, acc, acc_at)
        return (acc + s, g_sel, acc_at)

    _, g_sel, acc_at = lax.fori_loop(
        0, ngroups, gbody, (jnp.int32(0), jnp.int32(0), jnp.int32(0))
    )
    v = comb[pl.ds(g_sel * 16, 16)]
    inc = plsc.cumsum(v)
    total = jnp.sum(v)
    above = acc_at + (total - inc)
    hit = jnp.logical_and(above < k_need, above + v >= k_need)
    lane = lax.iota(jnp.int32, 16)
    b_sel = g_sel * 16 + jnp.sum(jnp.where(hit, lane, 0))
    k_at = k_need - jnp.sum(jnp.where(hit, above, 0))
    return b_sel, k_at


def _combine_workers(h_hbm, row, comb, nb):
    _zero_i32(comb, nb)

    def wbody(w, _):
        pltpu.sync_copy(h_hbm.at[pl.ds(pl.multiple_of(w * nb, nb), nb)], row)

        def gbody(g, _):
            base = g * 16
            comb[pl.ds(base, 16)] = comb[pl.ds(base, 16)] + row[pl.ds(base, 16)]
            return 0

        lax.fori_loop(0, nb // 16, gbody, 0)
        return 0

    lax.fori_loop(0, NW, wbody, 0)


def _combine_lanes(hist, nb, outb):
    def body(g, _):
        base = g * 16
        acc = hist[pl.ds(base, 16)]
        for l in range(1, L):
            acc = acc + hist[pl.ds(l * nb + base, 16)]
        outb[pl.ds(base, 16)] = acc
        return 0

    lax.fori_loop(0, nb // 16, body, 0)


def _hist_data_pass(x_hbm, buf, hist, wid, nb, bin_shift, bin_mask,
                    prefix_shift, prefix_val):
    lane_base = lax.iota(jnp.int32, 16) * nb
    ones = jnp.ones((16,), jnp.int32)

    def chunk_body(c, _):
        off = pl.multiple_of(wid * NPW + c * CHUNK, CHUNK)
        pltpu.sync_copy(x_hbm.at[pl.ds(off, CHUNK)], buf)

        def vbody(i, _):
            v = buf[pl.ds(i * 16, 16)]
            bits = lax.bitcast_convert_type(jnp.maximum(v, 0.0), jnp.int32)
            bin_ = jnp.bitwise_and(
                lax.shift_right_logical(bits, bin_shift), bin_mask
            )
            if prefix_shift is None:
                mask = None
            else:
                mask = lax.shift_right_logical(bits, prefix_shift) == prefix_val
            plsc.addupdate_scatter(hist, [lane_base + bin_], ones, mask=mask)
            return 0

        lax.fori_loop(0, CHUNK // 16, vbody, 0)
        return 0

    lax.fori_loop(0, NCHUNK, chunk_body, 0)


@functools.partial(
    pl.kernel,
    out_type=jax.ShapeDtypeStruct((NW, NB1), jnp.int32),
    mesh=_mesh,
    compiler_params=pltpu.CompilerParams(needs_layout_passes=False),
    scratch_types=[
        pltpu.VMEM((CHUNK,), jnp.float32),
        pltpu.VMEM((L * NB1,), jnp.int32),
        pltpu.VMEM((NB1,), jnp.int32),
    ],
)
def _hist1_kernel(x_hbm, out_hbm, buf, hist, outb):
    wid = lax.axis_index("s") * NC + lax.axis_index("c")
    _zero_i32(hist, L * NB1)
    _hist_data_pass(x_hbm, buf, hist, wid, NB1, 24, NB1 - 1, None, None)
    _combine_lanes(hist, NB1, outb)
    pltpu.sync_copy(outb, out_hbm.at[wid])


@functools.partial(
    pl.kernel,
    out_type=jax.ShapeDtypeStruct((NW, NB2), jnp.int32),
    mesh=_mesh,
    compiler_params=pltpu.CompilerParams(needs_layout_passes=False),
    scratch_types=[
        pltpu.VMEM((CHUNK,), jnp.float32),
        pltpu.VMEM((L * NB2,), jnp.int32),
        pltpu.VMEM((NB2,), jnp.int32),
        pltpu.VMEM((NB1,), jnp.int32),
        pltpu.VMEM((NB1,), jnp.int32),
    ],
)
def _hist2_kernel(x_hbm, h1_hbm, out_hbm, buf, hist, outb, row1, comb1):
    wid = lax.axis_index("s") * NC + lax.axis_index("c")
    _combine_workers(h1_hbm, row1, comb1, NB1)
    b1, _ = _select_hist(comb1, NB1, KTOT)
    _zero_i32(hist, L * NB2)
    _hist_data_pass(x_hbm, buf, hist, wid, NB2, 12, NB2 - 1, 24, b1)
    _combine_lanes(hist, NB2, outb)
    pltpu.sync_copy(outb, out_hbm.at[wid])


CCAP = 4096


@functools.partial(
    pl.kernel,
    out_type=(
        jax.ShapeDtypeStruct((NW, NB3), jnp.int32),
        jax.ShapeDtypeStruct((NW, CCAP), jnp.int32),
        jax.ShapeDtypeStruct((NW, CCAP), jnp.int32),
        jax.ShapeDtypeStruct((NW, 16), jnp.int32),
    ),
    mesh=_mesh,
    compiler_params=pltpu.CompilerParams(needs_layout_passes=False),
    scratch_types=[
        pltpu.VMEM((CHUNK,), jnp.float32),
        pltpu.VMEM((L * NB3,), jnp.int32),
        pltpu.VMEM((NB3,), jnp.int32),
        pltpu.VMEM((NB1,), jnp.int32),
        pltpu.VMEM((NB1,), jnp.int32),
        pltpu.VMEM((NB2,), jnp.int32),
        pltpu.VMEM((NB2,), jnp.int32),
        pltpu.VMEM((CCAP,), jnp.int32),
        pltpu.VMEM((CCAP,), jnp.int32),
        pltpu.VMEM((16,), jnp.int32),
    ],
)
def _hist3_kernel(x_hbm, h1_hbm, h2_hbm, out_hbm, ci_hbm, cb_hbm, cc_hbm,
                  buf, hist, outb, row1, comb1, row2, comb2, civ, cbv, cntv):
    wid = lax.axis_index("s") * NC + lax.axis_index("c")
    _combine_workers(h1_hbm, row1, comb1, NB1)
    b1, k1 = _select_hist(comb1, NB1, KTOT)
    _combine_workers(h2_hbm, row2, comb2, NB2)
    b2, _ = _select_hist(comb2, NB2, k1)
    p2 = b1 * NB2 + b2
    _zero_i32(hist, L * NB3)

    lane = lax.iota(jnp.int32, 16)
    lane_base = lane * NB3
    ones = jnp.ones((16,), jnp.int32)

    def chunk_body(c, cursor):
        off = pl.multiple_of(wid * NPW + c * CHUNK, CHUNK)
        pltpu.sync_copy(x_hbm.at[pl.ds(off, CHUNK)], buf)

        def vbody(i, cursor):
            v = buf[pl.ds(i * 16, 16)]
            bits = lax.bitcast_convert_type(jnp.maximum(v, 0.0), jnp.int32)
            mask = lax.shift_right_logical(bits, 12) == p2
            bin_ = jnp.bitwise_and(bits, NB3 - 1)
            plsc.addupdate_scatter(hist, [lane_base + bin_], ones, mask=mask)
            rank = plsc.cumsum(jnp.where(mask, jnp.int32(1), jnp.int32(0)))
            slot = jnp.minimum(cursor + rank - 1, CCAP - 1)
            idxv = off + i * 16 + lane
            plsc.store_scatter(civ, [slot], idxv, mask=mask)
            plsc.store_scatter(cbv, [slot], bits, mask=mask)
            return cursor + rank[15]

        return lax.fori_loop(0, CHUNK // 16, vbody, cursor)

    cnt = lax.fori_loop(0, NCHUNK, chunk_body, jnp.int32(0))
    _combine_lanes(hist, NB3, outb)
    pltpu.sync_copy(outb, out_hbm.at[wid])
    pltpu.sync_copy(civ, ci_hbm.at[wid])
    pltpu.sync_copy(cbv, cb_hbm.at[wid])
    cntv[...] = jnp.zeros((16,), jnp.int32) + jnp.minimum(cnt, CCAP)
    pltpu.sync_copy(cntv, cc_hbm.at[wid])


@functools.partial(
    pl.kernel,
    out_type=jax.ShapeDtypeStruct((NW, 16), jnp.int32),
    mesh=_mesh,
    compiler_params=pltpu.CompilerParams(needs_layout_passes=False),
    scratch_types=[
        pltpu.VMEM((NB1,), jnp.int32),
        pltpu.VMEM((NB1,), jnp.int32),
        pltpu.VMEM((NB2,), jnp.int32),
        pltpu.VMEM((NB2,), jnp.int32),
        pltpu.VMEM((NB3,), jnp.int32),
        pltpu.VMEM((NB3,), jnp.int32),
        pltpu.VMEM((16,), jnp.int32),
        pltpu.VMEM((CCAP,), jnp.int32),
        pltpu.VMEM((CCAP,), jnp.int32),
        pltpu.VMEM((16,), jnp.int32),
        pltpu.VMEM((16,), jnp.int32),
    ],
)
def _thresh_kernel(h1_hbm, h2_hbm, h3_hbm, ci_hbm, cb_hbm, cc_hbm, out_hbm,
                   row1, comb1, row2, comb2, row3, comb3, colrow, civ, cbv,
                   ccv, tout):
    wid = lax.axis_index("s") * NC + lax.axis_index("c")
    lane = lax.iota(jnp.int32, 16)

    _combine_workers(h1_hbm, row1, comb1, NB1)
    b1, k1 = _select_hist(comb1, NB1, KTOT)
    _combine_workers(h2_hbm, row2, comb2, NB2)
    b2, k2 = _select_hist(comb2, NB2, k1)
    _combine_workers(h3_hbm, row3, comb3, NB3)
    b3, m = _select_hist(comb3, NB3, k2)
    t = b1 * (1 << 24) + b2 * (1 << 12) + b3

    base3 = pl.multiple_of(b3 - jnp.bitwise_and(b3, 15), 16)
    lane_sel = jnp.bitwise_and(b3, 15)

    def colbody(w, carry):
        prior, own = carry
        pltpu.sync_copy(h3_hbm.at[pl.ds(w * NB3 + base3, 16)], colrow)
        cw = jnp.sum(jnp.where(lane == lane_sel, colrow[...], jnp.int32(0)))
        prior = prior + jnp.where(w < wid, cw, jnp.int32(0))
        own = own + jnp.where(w == wid, cw, jnp.int32(0))
        return prior, own

    prior, own = lax.fori_loop(0, NW, colbody, (jnp.int32(0), jnp.int32(0)))
    r_w = m - prior
    exists = jnp.logical_and(r_w >= 1, r_w <= own)

    pltpu.sync_copy(ci_hbm.at[wid], civ)
    pltpu.sync_copy(cb_hbm.at[wid], cbv)
    pltpu.sync_copy(cc_hbm.at[wid], ccv)
    ncand = ccv[...][0]

    def scanbody(j, carry):
        val, cnt_before = carry
        cb = cbv[pl.ds(j * 16, 16)]
        ci = civ[pl.ds(j * 16, 16)]
        ordv = j * 16 + lane
        mask = jnp.logical_and(cb == t, ordv < ncand)
        rank = plsc.cumsum(jnp.where(mask, jnp.int32(1), jnp.int32(0)))
        hit = jnp.logical_and(mask, cnt_before + rank == r_w)
        val = val + jnp.sum(jnp.where(hit, ci, jnp.int32(0)))
        return val, cnt_before + rank[15]

    val, _ = lax.fori_loop(0, CCAP // 16, scanbody,
                           (jnp.int32(0), jnp.int32(0)))
    val = jnp.where(exists, val, jnp.int32(0))
    tout[...] = (jnp.where(lane == 0, t, jnp.int32(0))
                 + jnp.where(lane == 1, val, jnp.int32(0)))
    pltpu.sync_copy(tout, out_hbm.at[wid])


BLK_R = 128


def _finalize_body(t_ref, x_ref, o_ref):
    t = t_ref[0, 0]
    im = t_ref[0, 1]
    for w in range(1, NW):
        im = im + t_ref[w, 1]
    acts = jnp.maximum(x_ref[...], 0.0)
    bits = lax.bitcast_convert_type(acts, jnp.int32)
    ridx = lax.broadcasted_iota(jnp.int32, (BLK_R, COLS), 0)
    cidx = lax.broadcasted_iota(jnp.int32, (BLK_R, COLS), 1)
    flat = (pl.program_id(0) * BLK_R + ridx) * COLS + cidx
    keep = jnp.logical_or(
        bits > t, jnp.logical_and(bits == t, flat <= im)
    )
    o_ref[...] = jnp.where(keep, acts, 0.0)


def _finalize(tmeta, x):
    return pl.pallas_call(
        _finalize_body,
        grid=(ROWS // BLK_R,),
        in_specs=[
            pl.BlockSpec(memory_space=pltpu.SMEM),
            pl.BlockSpec((BLK_R, COLS), lambda i: (i, 0)),
        ],
        out_specs=pl.BlockSpec((BLK_R, COLS), lambda i: (i, 0)),
        out_shape=jax.ShapeDtypeStruct((ROWS, COLS), jnp.float32),
    )(tmeta, x)


def kernel(x):
    xf = x.reshape(-1)
    h1 = _hist1_kernel(xf).reshape(-1)
    h2 = _hist2_kernel(xf, h1).reshape(-1)
    h3, ci, cb, cc = _hist3_kernel(xf, h1, h2)
    tmeta = _thresh_kernel(h1, h2, h3.reshape(-1), ci, cb, cc)
    return _finalize(tmeta, x)


# speedup vs baseline: 1.2188x; 1.2188x over previous
import functools

import jax
import jax.numpy as jnp
from jax import lax
from jax.experimental import pallas as pl
from jax.experimental.pallas import tpu as pltpu
from jax.experimental.pallas import tpu_sc as plsc

NC, NS, L = 2, 16, 16
NW = NC * NS

ROWS, COLS = 2048, 16384
N = ROWS * COLS
KTOT = 32 * ROWS

NPW = N // NW

NB1 = 256
NB2 = 4096
NB3 = 4096
CCAP = 4096

CHUNK1 = 32768
CHUNK2 = 16384
CHUNK3 = 8192

_mesh = plsc.VectorSubcoreMesh(
    core_axis_name="c", subcore_axis_name="s", num_cores=NC, num_subcores=NS
)
_params = pltpu.CompilerParams(needs_layout_passes=False)


def _zero_i32(ref, n):
    z = jnp.zeros((16,), jnp.int32)

    def body(i, _):
        ref[pl.ds(i * 16, 16)] = z
        return 0

    lax.fori_loop(0, n // 16, body, 0, unroll=8)


def _select_hist(comb, nb, k_need):
    ngroups = nb // 16

    def gbody(i, carry):
        acc, g_sel, acc_at = carry
        g = ngroups - 1 - i
        v = comb[pl.ds(g * 16, 16)]
        s = jnp.sum(v)
        hit = jnp.logical_and(acc < k_need, acc + s >= k_need)
        g_sel = jnp.where(hit, g, g_sel)
        acc_at = jnp.where(hit, acc, acc_at)
        return (acc + s, g_sel, acc_at)

    _, g_sel, acc_at = lax.fori_loop(
        0, ngroups, gbody, (jnp.int32(0), jnp.int32(0), jnp.int32(0))
    )
    v = comb[pl.ds(g_sel * 16, 16)]
    inc = plsc.cumsum(v)
    total = jnp.sum(v)
    above = acc_at + (total - inc)
    hit = jnp.logical_and(above < k_need, above + v >= k_need)
    lane = lax.iota(jnp.int32, 16)
    b_sel = g_sel * 16 + jnp.sum(jnp.where(hit, lane, 0))
    k_at = k_need - jnp.sum(jnp.where(hit, above, 0))
    return b_sel, k_at


def _combine_bulk(h_hbm, stage, comb, nb, rpb):
    _zero_i32(comb, nb)

    def bbody(bk, _):
        off = pl.multiple_of(bk * rpb * nb, 8)
        pltpu.sync_copy(h_hbm.at[pl.ds(off, rpb * nb)], stage)

        def gbody(g, _):
            base = g * 16
            acc = comb[pl.ds(base, 16)]
            for r in range(rpb):
                acc = acc + stage[pl.ds(r * nb + base, 16)]
            comb[pl.ds(base, 16)] = acc
            return 0

        lax.fori_loop(0, nb // 16, gbody, 0, unroll=4)
        return 0

    lax.fori_loop(0, NW // rpb, bbody, 0)


def _combine_lanes(hist, nb, outb):
    def body(g, _):
        base = g * 16
        acc = hist[pl.ds(base, 16)]
        for l in range(1, L):
            acc = acc + hist[pl.ds(l * nb + base, 16)]
        outb[pl.ds(base, 16)] = acc
        return 0

    lax.fori_loop(0, nb // 16, body, 0)


def _stream_chunks(x_hbm, buf0, buf1, sem0, sem1, wid, chunk, process, init):
    base = wid * NPW
    npair = NPW // chunk // 2

    pltpu.async_copy(x_hbm.at[pl.ds(pl.multiple_of(base, 8), chunk)],
                     buf0, sem0)

    def pair(p, carry):
        off0 = pl.multiple_of(base + (2 * p) * chunk, 8)
        off1 = pl.multiple_of(base + (2 * p + 1) * chunk, 8)
        pltpu.async_copy(x_hbm.at[pl.ds(off1, chunk)], buf1, sem1)
        pltpu.make_async_copy(
            x_hbm.at[pl.ds(off0, chunk)], buf0, sem0).wait()
        carry = process(buf0, off0, carry)

        @pl.when(p < npair - 1)
        def _():
            off2 = pl.multiple_of(base + (2 * p + 2) * chunk, 8)
            pltpu.async_copy(x_hbm.at[pl.ds(off2, chunk)], buf0, sem0)

        pltpu.make_async_copy(
            x_hbm.at[pl.ds(off1, chunk)], buf1, sem1).wait()
        carry = process(buf1, off1, carry)
        return carry

    return lax.fori_loop(0, npair, pair, init)


def _mk_hist_process(hist, nb, chunk, bin_shift, prefix_shift, prefix_val):
    lane_base = lax.iota(jnp.int32, 16) * nb
    ones = jnp.ones((16,), jnp.int32)

    def process(buf, off, carry):
        def vbody(i, c):
            v = buf[pl.ds(i * 16, 16)]
            bits = lax.bitcast_convert_type(v, jnp.int32)
            bin_ = lax.shift_right_logical(bits, bin_shift)
            if nb - 1 != (1 << (32 - bin_shift)) - 1:
                bin_ = jnp.bitwise_and(bin_, nb - 1)
            if prefix_shift is None:
                mask = None
            else:
                mask = lax.shift_right_logical(bits, prefix_shift) == prefix_val
            plsc.addupdate_scatter(hist, [lane_base + bin_], ones, mask=mask)
            return c

        return lax.fori_loop(0, chunk // 16, vbody, carry, unroll=8)

    return process


@functools.partial(
    pl.kernel,
    out_type=jax.ShapeDtypeStruct((NW, NB1), jnp.int32),
    mesh=_mesh,
    compiler_params=_params,
    scratch_types=[
        pltpu.VMEM((CHUNK1,), jnp.float32),
        pltpu.VMEM((CHUNK1,), jnp.float32),
        pltpu.VMEM((L * NB1,), jnp.int32),
        pltpu.VMEM((NB1,), jnp.int32),
        pltpu.SemaphoreType.DMA,
        pltpu.SemaphoreType.DMA,
    ],
)
def _hist1_kernel(x_hbm, out_hbm, buf0, buf1, hist, outb, sem0, sem1):
    wid = lax.axis_index("s") * NC + lax.axis_index("c")
    _zero_i32(hist, L * NB1)
    proc = _mk_hist_process(hist, NB1, CHUNK1, 24, None, None)
    _stream_chunks(x_hbm, buf0, buf1, sem0, sem1, wid, CHUNK1, proc,
                   jnp.int32(0))
    _combine_lanes(hist, NB1, outb)
    pltpu.sync_copy(outb, out_hbm.at[wid])


@functools.partial(
    pl.kernel,
    out_type=jax.ShapeDtypeStruct((NW, NB2), jnp.int32),
    mesh=_mesh,
    compiler_params=_params,
    scratch_types=[
        pltpu.VMEM((CHUNK2,), jnp.float32),
        pltpu.VMEM((CHUNK2,), jnp.float32),
        pltpu.VMEM((L * NB2,), jnp.int32),
        pltpu.VMEM((NB2,), jnp.int32),
        pltpu.VMEM((NW * NB1,), jnp.int32),
        pltpu.VMEM((NB1,), jnp.int32),
        pltpu.SemaphoreType.DMA,
        pltpu.SemaphoreType.DMA,
    ],
)
def _hist2_kernel(x_hbm, h1_hbm, out_hbm, buf0, buf1, hist, outb, stage1,
                  comb1, sem0, sem1):
    wid = lax.axis_index("s") * NC + lax.axis_index("c")
    _combine_bulk(h1_hbm, stage1, comb1, NB1, NW)
    b1, _ = _select_hist(comb1, 128, KTOT)
    _zero_i32(hist, L * NB2)
    proc = _mk_hist_process(hist, NB2, CHUNK2, 12, 24, b1)
    _stream_chunks(x_hbm, buf0, buf1, sem0, sem1, wid, CHUNK2, proc,
                   jnp.int32(0))
    _combine_lanes(hist, NB2, outb)
    pltpu.sync_copy(outb, out_hbm.at[wid])


@functools.partial(
    pl.kernel,
    out_type=(
        jax.ShapeDtypeStruct((NW, NB3), jnp.int32),
        jax.ShapeDtypeStruct((NW, CCAP), jnp.int32),
        jax.ShapeDtypeStruct((NW, CCAP), jnp.int32),
        jax.ShapeDtypeStruct((NW, 16), jnp.int32),
    ),
    mesh=_mesh,
    compiler_params=_params,
    scratch_types=[
        pltpu.VMEM((CHUNK3,), jnp.float32),
        pltpu.VMEM((CHUNK3,), jnp.float32),
        pltpu.VMEM((L * NB3,), jnp.int32),
        pltpu.VMEM((NB3,), jnp.int32),
        pltpu.VMEM((NW * NB1,), jnp.int32),
        pltpu.VMEM((NB1,), jnp.int32),
        pltpu.VMEM((4 * NB2,), jnp.int32),
        pltpu.VMEM((NB2,), jnp.int32),
        pltpu.VMEM((CCAP,), jnp.int32),
        pltpu.VMEM((CCAP,), jnp.int32),
        pltpu.VMEM((16,), jnp.int32),
        pltpu.SemaphoreType.DMA,
        pltpu.SemaphoreType.DMA,
    ],
)
def _hist3_kernel(x_hbm, h1_hbm, h2_hbm, out_hbm, ci_hbm, cb_hbm, cc_hbm,
                  buf0, buf1, hist, outb, stage1, comb1, stage2, comb2,
                  civ, cbv, cntv, sem0, sem1):
    wid = lax.axis_index("s") * NC + lax.axis_index("c")
    _combine_bulk(h1_hbm, stage1, comb1, NB1, NW)
    b1, k1 = _select_hist(comb1, 128, KTOT)
    _combine_bulk(h2_hbm, stage2, comb2, NB2, 4)
    b2, _ = _select_hist(comb2, NB2, k1)
    p2 = b1 * NB2 + b2
    _zero_i32(hist, L * NB3)

    lane = lax.iota(jnp.int32, 16)
    lane_base = lane * NB3
    ones = jnp.ones((16,), jnp.int32)

    def process(buf, off, cursor):
        def vbody(i, cursor):
            v = buf[pl.ds(i * 16, 16)]
            bits = lax.bitcast_convert_type(v, jnp.int32)
            mask = lax.shift_right_logical(bits, 12) == p2
            bin_ = jnp.bitwise_and(bits, NB3 - 1)
            plsc.addupdate_scatter(hist, [lane_base + bin_], ones, mask=mask)
            rank = plsc.cumsum(jnp.where(mask, jnp.int32(1), jnp.int32(0)))
            slot = jnp.minimum(cursor + rank - 1, CCAP - 1)
            idxv = off + i * 16 + lane
            plsc.store_scatter(civ, [slot], idxv, mask=mask)
            plsc.store_scatter(cbv, [slot], bits, mask=mask)
            return cursor + rank[15]

        return lax.fori_loop(0, CHUNK3 // 16, vbody, cursor, unroll=4)

    cnt = _stream_chunks(x_hbm, buf0, buf1, sem0, sem1, wid, CHUNK3, process,
                         jnp.int32(0))
    _combine_lanes(hist, NB3, outb)
    pltpu.sync_copy(outb, out_hbm.at[wid])
    pltpu.sync_copy(civ, ci_hbm.at[wid])
    pltpu.sync_copy(cbv, cb_hbm.at[wid])
    cntv[...] = jnp.zeros((16,), jnp.int32) + jnp.minimum(cnt, CCAP)
    pltpu.sync_copy(cntv, cc_hbm.at[wid])


@functools.partial(
    pl.kernel,
    out_type=jax.ShapeDtypeStruct((NW, 16), jnp.int32),
    mesh=_mesh,
    compiler_params=_params,
    scratch_types=[
        pltpu.VMEM((NW * NB1,), jnp.int32),
        pltpu.VMEM((NB1,), jnp.int32),
        pltpu.VMEM((4 * NB2,), jnp.int32),
        pltpu.VMEM((NB2,), jnp.int32),
        pltpu.VMEM((NB3,), jnp.int32),
        pltpu.VMEM((16,), jnp.int32),
        pltpu.VMEM((16,), jnp.int32),
        pltpu.VMEM((CCAP,), jnp.int32),
        pltpu.VMEM((CCAP,), jnp.int32),
        pltpu.VMEM((16,), jnp.int32),
        pltpu.VMEM((16,), jnp.int32),
        pltpu.SemaphoreType.DMA,
    ],
)
def _thresh_kernel(h1_hbm, h2_hbm, h3_hbm, ci_hbm, cb_hbm, cc_hbm, out_hbm,
                   stage1, comb1, stage2, comb3, comb3b, col0, col1, civ,
                   cbv, ccv, tout, sem):
    wid = lax.axis_index("s") * NC + lax.axis_index("c")
    lane = lax.iota(jnp.int32, 16)

    _combine_bulk(h1_hbm, stage1, comb1, NB1, NW)
    b1, k1 = _select_hist(comb1, 128, KTOT)
    _combine_bulk(h2_hbm, stage2, comb3, NB2, 4)
    b2, k2 = _select_hist(comb3, NB2, k1)
    _combine_bulk(h3_hbm, stage2, comb3b, NB3, 4)
    b3, m = _select_hist(comb3b, NB3, k2)
    t = b1 * (1 << 24) + b2 * (1 << 12) + b3

    idx0 = lane * NB3 + b3
    pltpu.async_copy(h3_hbm.at[idx0], col0, sem).wait()
    pltpu.async_copy(h3_hbm.at[idx0 + 16 * NB3], col1, sem).wait()
    c0 = col0[...]
    c1 = col1[...]
    prior = (jnp.sum(jnp.where(lane < wid, c0, jnp.int32(0)))
             + jnp.sum(jnp.where(lane + 16 < wid, c1, jnp.int32(0))))
    own = (jnp.sum(jnp.where(lane == wid, c0, jnp.int32(0)))
           + jnp.sum(jnp.where(lane + 16 == wid, c1, jnp.int32(0))))
    r_w = m - prior
    exists = jnp.logical_and(r_w >= 1, r_w <= own)

    pltpu.sync_copy(ci_hbm.at[wid], civ)
    pltpu.sync_copy(cb_hbm.at[wid], cbv)
    pltpu.sync_copy(cc_hbm.at[wid], ccv)
    ncand = ccv[...][0]

    def scanbody(j, carry):
        val, cnt_before = carry
        cb = cbv[pl.ds(j * 16, 16)]
        ci = civ[pl.ds(j * 16, 16)]
        ordv = j * 16 + lane
        mask = jnp.logical_and(cb == t, ordv < ncand)
        rank = plsc.cumsum(jnp.where(mask, jnp.int32(1), jnp.int32(0)))
        hit = jnp.logical_and(mask, cnt_before + rank == r_w)
        val = val + jnp.sum(jnp.where(hit, ci, jnp.int32(0)))
        return val, cnt_before + rank[15]

    nit = lax.shift_right_logical(ncand + 15, 4)
    val, _ = lax.fori_loop(0, nit, scanbody, (jnp.int32(0), jnp.int32(0)))
    val = jnp.where(exists, val, jnp.int32(0))
    tout[...] = (jnp.where(lane == 0, t, jnp.int32(0))
                 + jnp.where(lane == 1, val, jnp.int32(0)))
    pltpu.sync_copy(tout, out_hbm.at[wid])


BLK_R = 128


def _finalize_body(t_ref, x_ref, o_ref):
    t = t_ref[0, 0]
    im = t_ref[0, 1]
    for w in range(1, NW):
        im = im + t_ref[w, 1]
    acts = jnp.maximum(x_ref[...], 0.0)
    bits = lax.bitcast_convert_type(acts, jnp.int32)
    ridx = lax.broadcasted_iota(jnp.int32, (BLK_R, COLS), 0)
    cidx = lax.broadcasted_iota(jnp.int32, (BLK_R, COLS), 1)
    flat = (pl.program_id(0) * BLK_R + ridx) * COLS + cidx
    keep = jnp.logical_or(
        bits > t, jnp.logical_and(bits == t, flat <= im)
    )
    o_ref[...] = jnp.where(keep, acts, 0.0)


def _finalize(tmeta, x):
    return pl.pallas_call(
        _finalize_body,
        grid=(ROWS // BLK_R,),
        in_specs=[
            pl.BlockSpec(memory_space=pltpu.SMEM),
            pl.BlockSpec((BLK_R, COLS), lambda i: (i, 0)),
        ],
        out_specs=pl.BlockSpec((BLK_R, COLS), lambda i: (i, 0)),
        out_shape=jax.ShapeDtypeStruct((ROWS, COLS), jnp.float32),
    )(tmeta, x)


def kernel(x):
    xf = x.reshape(-1)
    h1 = _hist1_kernel(xf).reshape(-1)
    h2 = _hist2_kernel(xf, h1).reshape(-1)
    h3, ci, cb, cc = _hist3_kernel(xf, h1, h2)
    tmeta = _thresh_kernel(h1, h2, h3.reshape(-1), ci, cb, cc)
    return _finalize(tmeta, x)


# speedup vs baseline: 1.3673x; 1.1218x over previous
import functools

import jax
import jax.numpy as jnp
from jax import lax
from jax.experimental import pallas as pl
from jax.experimental.pallas import tpu as pltpu
from jax.experimental.pallas import tpu_sc as plsc

NC, NS, L = 2, 16, 16
NW = NC * NS

ROWS, COLS = 2048, 16384
N = ROWS * COLS
KTOT = 32 * ROWS

NPW = N // NW

NB1 = 256
NB2 = 4096
NB3 = 4096
CCAP = 4096

CHUNK1 = 32768
CHUNK2 = 16384
CHUNK3 = 8192

_mesh = plsc.VectorSubcoreMesh(
    core_axis_name="c", subcore_axis_name="s", num_cores=NC, num_subcores=NS
)
_params = pltpu.CompilerParams(needs_layout_passes=False)


def _zero_i32(ref, n):
    z = jnp.zeros((16,), jnp.int32)

    def body(i, _):
        ref[pl.ds(i * 16, 16)] = z
        return 0

    lax.fori_loop(0, n // 16, body, 0, unroll=8)


def _select_hist(comb, nb, k_need):
    ngroups = nb // 16

    def gbody(i, carry):
        acc, g_sel, acc_at = carry
        g = ngroups - 1 - i
        v = comb[pl.ds(g * 16, 16)]
        s = jnp.sum(v)
        hit = jnp.logical_and(acc < k_need, acc + s >= k_need)
        g_sel = jnp.where(hit, g, g_sel)
        acc_at = jnp.where(hit, acc, acc_at)
        return (acc + s, g_sel, acc_at)

    _, g_sel, acc_at = lax.fori_loop(
        0, ngroups, gbody, (jnp.int32(0), jnp.int32(0), jnp.int32(0))
    )
    v = comb[pl.ds(g_sel * 16, 16)]
    inc = plsc.cumsum(v)
    total = jnp.sum(v)
    above = acc_at + (total - inc)
    hit = jnp.logical_and(above < k_need, above + v >= k_need)
    lane = lax.iota(jnp.int32, 16)
    b_sel = g_sel * 16 + jnp.sum(jnp.where(hit, lane, 0))
    k_at = k_need - jnp.sum(jnp.where(hit, above, 0))
    return b_sel, k_at


def _combine_bulk(h_hbm, stage, comb, nb, rpb):
    _zero_i32(comb, nb)

    def bbody(bk, _):
        off = pl.multiple_of(bk * rpb * nb, 8)
        pltpu.sync_copy(h_hbm.at[pl.ds(off, rpb * nb)], stage)

        def gbody(g, _):
            base = g * 16
            acc = comb[pl.ds(base, 16)]
            for r in range(rpb):
                acc = acc + stage[pl.ds(r * nb + base, 16)]
            comb[pl.ds(base, 16)] = acc
            return 0

        lax.fori_loop(0, nb // 16, gbody, 0, unroll=4)
        return 0

    lax.fori_loop(0, NW // rpb, bbody, 0)


def _combine_lanes(hist, nb, outb):
    lane = lax.iota(jnp.int32, 16)

    def body(g, _):
        base = g * 256
        acc = jnp.zeros((16,), jnp.int32)
        for l in range(L):
            acc = acc + plsc.load_gather(hist, [base + lane * 16 + l])
        outb[pl.ds(g * 16, 16)] = acc
        return 0

    lax.fori_loop(0, nb // 16, body, 0)


def _stream_chunks(x_hbm, buf0, buf1, sem0, sem1, wid, chunk, process, init):
    base = wid * NPW
    npair = NPW // chunk // 2

    pltpu.async_copy(x_hbm.at[pl.ds(pl.multiple_of(base, 8), chunk)],
                     buf0, sem0)

    def pair(p, carry):
        off0 = pl.multiple_of(base + (2 * p) * chunk, 8)
        off1 = pl.multiple_of(base + (2 * p + 1) * chunk, 8)
        pltpu.async_copy(x_hbm.at[pl.ds(off1, chunk)], buf1, sem1)
        pltpu.make_async_copy(
            x_hbm.at[pl.ds(off0, chunk)], buf0, sem0).wait()
        carry = process(buf0, off0, carry)

        @pl.when(p < npair - 1)
        def _():
            off2 = pl.multiple_of(base + (2 * p + 2) * chunk, 8)
            pltpu.async_copy(x_hbm.at[pl.ds(off2, chunk)], buf0, sem0)

        pltpu.make_async_copy(
            x_hbm.at[pl.ds(off1, chunk)], buf1, sem1).wait()
        carry = process(buf1, off1, carry)
        return carry

    return lax.fori_loop(0, npair, pair, init)


def _mk_hist_process(hist, nb, chunk, bin_shift, prefix_shift, prefix_val):
    lane = lax.iota(jnp.int32, 16)
    ones = jnp.ones((16,), jnp.int32)

    def process(buf, off, carry):
        def vbody(i, c):
            v = buf[pl.ds(i * 16, 16)]
            bits = lax.bitcast_convert_type(v, jnp.int32)
            bin_ = lax.shift_right_logical(bits, bin_shift)
            if nb - 1 != (1 << (32 - bin_shift)) - 1:
                bin_ = jnp.bitwise_and(bin_, nb - 1)
            if prefix_shift is None:
                mask = None
            else:
                mask = lax.shift_right_logical(bits, prefix_shift) == prefix_val
            plsc.addupdate_scatter(hist, [bin_ * 16 + lane], ones, mask=mask)
            return c

        return lax.fori_loop(0, chunk // 16, vbody, carry, unroll=8)

    return process


@functools.partial(
    pl.kernel,
    out_type=jax.ShapeDtypeStruct((NW, NB1), jnp.int32),
    mesh=_mesh,
    compiler_params=_params,
    scratch_types=[
        pltpu.VMEM((CHUNK1,), jnp.float32),
        pltpu.VMEM((CHUNK1,), jnp.float32),
        pltpu.VMEM((L * NB1,), jnp.int32),
        pltpu.VMEM((NB1,), jnp.int32),
        pltpu.SemaphoreType.DMA,
        pltpu.SemaphoreType.DMA,
    ],
)
def _hist1_kernel(x_hbm, out_hbm, buf0, buf1, hist, outb, sem0, sem1):
    wid = lax.axis_index("s") * NC + lax.axis_index("c")
    _zero_i32(hist, L * NB1)
    proc = _mk_hist_process(hist, NB1, CHUNK1, 24, None, None)
    _stream_chunks(x_hbm, buf0, buf1, sem0, sem1, wid, CHUNK1, proc,
                   jnp.int32(0))
    _combine_lanes(hist, NB1, outb)
    pltpu.sync_copy(outb, out_hbm.at[wid])


@functools.partial(
    pl.kernel,
    out_type=jax.ShapeDtypeStruct((NW, NB2), jnp.int32),
    mesh=_mesh,
    compiler_params=_params,
    scratch_types=[
        pltpu.VMEM((CHUNK2,), jnp.float32),
        pltpu.VMEM((CHUNK2,), jnp.float32),
        pltpu.VMEM((L * NB2,), jnp.int32),
        pltpu.VMEM((NB2,), jnp.int32),
        pltpu.VMEM((NW * NB1,), jnp.int32),
        pltpu.VMEM((NB1,), jnp.int32),
        pltpu.SemaphoreType.DMA,
        pltpu.SemaphoreType.DMA,
    ],
)
def _hist2_kernel(x_hbm, h1_hbm, out_hbm, buf0, buf1, hist, outb, stage1,
                  comb1, sem0, sem1):
    wid = lax.axis_index("s") * NC + lax.axis_index("c")
    _combine_bulk(h1_hbm, stage1, comb1, NB1, NW)
    b1, _ = _select_hist(comb1, 128, KTOT)
    _zero_i32(hist, L * NB2)
    proc = _mk_hist_process(hist, NB2, CHUNK2, 12, 24, b1)
    _stream_chunks(x_hbm, buf0, buf1, sem0, sem1, wid, CHUNK2, proc,
                   jnp.int32(0))
    _combine_lanes(hist, NB2, outb)
    pltpu.sync_copy(outb, out_hbm.at[wid])


@functools.partial(
    pl.kernel,
    out_type=(
        jax.ShapeDtypeStruct((NW, NB3), jnp.int32),
        jax.ShapeDtypeStruct((NW, CCAP), jnp.int32),
        jax.ShapeDtypeStruct((NW, CCAP), jnp.int32),
        jax.ShapeDtypeStruct((NW, 16), jnp.int32),
    ),
    mesh=_mesh,
    compiler_params=_params,
    scratch_types=[
        pltpu.VMEM((CHUNK3,), jnp.float32),
        pltpu.VMEM((CHUNK3,), jnp.float32),
        pltpu.VMEM((L * NB3,), jnp.int32),
        pltpu.VMEM((NB3,), jnp.int32),
        pltpu.VMEM((NW * NB1,), jnp.int32),
        pltpu.VMEM((NB1,), jnp.int32),
        pltpu.VMEM((4 * NB2,), jnp.int32),
        pltpu.VMEM((NB2,), jnp.int32),
        pltpu.VMEM((CCAP,), jnp.int32),
        pltpu.VMEM((CCAP,), jnp.int32),
        pltpu.VMEM((16,), jnp.int32),
        pltpu.SemaphoreType.DMA,
        pltpu.SemaphoreType.DMA,
    ],
)
def _hist3_kernel(x_hbm, h1_hbm, h2_hbm, out_hbm, ci_hbm, cb_hbm, cc_hbm,
                  buf0, buf1, hist, outb, stage1, comb1, stage2, comb2,
                  civ, cbv, cntv, sem0, sem1):
    wid = lax.axis_index("s") * NC + lax.axis_index("c")
    _combine_bulk(h1_hbm, stage1, comb1, NB1, NW)
    b1, k1 = _select_hist(comb1, 128, KTOT)
    _combine_bulk(h2_hbm, stage2, comb2, NB2, 4)
    b2, _ = _select_hist(comb2, NB2, k1)
    p2 = b1 * NB2 + b2
    _zero_i32(hist, L * NB3)

    lane = lax.iota(jnp.int32, 16)
    ones = jnp.ones((16,), jnp.int32)

    def process(buf, off, cursor):
        def vbody(i, cursor):
            v = buf[pl.ds(i * 16, 16)]
            bits = lax.bitcast_convert_type(v, jnp.int32)
            mask = lax.shift_right_logical(bits, 12) == p2
            bin_ = jnp.bitwise_and(bits, NB3 - 1)
            plsc.addupdate_scatter(hist, [bin_ * 16 + lane], ones, mask=mask)
            rank = plsc.cumsum(jnp.where(mask, jnp.int32(1), jnp.int32(0)))
            slot = jnp.minimum(cursor + rank - 1, CCAP - 1)
            idxv = off + i * 16 + lane
            plsc.store_scatter(civ, [slot], idxv, mask=mask)
            plsc.store_scatter(cbv, [slot], bits, mask=mask)
            return cursor + rank[15]

        return lax.fori_loop(0, CHUNK3 // 16, vbody, cursor, unroll=4)

    cnt = _stream_chunks(x_hbm, buf0, buf1, sem0, sem1, wid, CHUNK3, process,
                         jnp.int32(0))
    _combine_lanes(hist, NB3, outb)
    pltpu.sync_copy(outb, out_hbm.at[wid])
    pltpu.sync_copy(civ, ci_hbm.at[wid])
    pltpu.sync_copy(cbv, cb_hbm.at[wid])
    cntv[...] = jnp.zeros((16,), jnp.int32) + jnp.minimum(cnt, CCAP)
    pltpu.sync_copy(cntv, cc_hbm.at[wid])


@functools.partial(
    pl.kernel,
    out_type=jax.ShapeDtypeStruct((NW, 16), jnp.int32),
    mesh=_mesh,
    compiler_params=_params,
    scratch_types=[
        pltpu.VMEM((NW * NB1,), jnp.int32),
        pltpu.VMEM((NB1,), jnp.int32),
        pltpu.VMEM((4 * NB2,), jnp.int32),
        pltpu.VMEM((NB2,), jnp.int32),
        pltpu.VMEM((NB3,), jnp.int32),
        pltpu.VMEM((16,), jnp.int32),
        pltpu.VMEM((16,), jnp.int32),
        pltpu.VMEM((CCAP,), jnp.int32),
        pltpu.VMEM((CCAP,), jnp.int32),
        pltpu.VMEM((16,), jnp.int32),
        pltpu.VMEM((16,), jnp.int32),
        pltpu.SemaphoreType.DMA,
    ],
)
def _thresh_kernel(h1_hbm, h2_hbm, h3_hbm, ci_hbm, cb_hbm, cc_hbm, out_hbm,
                   stage1, comb1, stage2, comb3, comb3b, col0, col1, civ,
                   cbv, ccv, tout, sem):
    wid = lax.axis_index("s") * NC + lax.axis_index("c")
    lane = lax.iota(jnp.int32, 16)

    _combine_bulk(h1_hbm, stage1, comb1, NB1, NW)
    b1, k1 = _select_hist(comb1, 128, KTOT)
    _combine_bulk(h2_hbm, stage2, comb3, NB2, 4)
    b2, k2 = _select_hist(comb3, NB2, k1)
    _combine_bulk(h3_hbm, stage2, comb3b, NB3, 4)
    b3, m = _select_hist(comb3b, NB3, k2)
    t = b1 * (1 << 24) + b2 * (1 << 12) + b3

    idx0 = lane * NB3 + b3
    pltpu.async_copy(h3_hbm.at[idx0], col0, sem).wait()
    pltpu.async_copy(h3_hbm.at[idx0 + 16 * NB3], col1, sem).wait()
    c0 = col0[...]
    c1 = col1[...]
    prior = (jnp.sum(jnp.where(lane < wid, c0, jnp.int32(0)))
             + jnp.sum(jnp.where(lane + 16 < wid, c1, jnp.int32(0))))
    own = (jnp.sum(jnp.where(lane == wid, c0, jnp.int32(0)))
           + jnp.sum(jnp.where(lane + 16 == wid, c1, jnp.int32(0))))
    r_w = m - prior
    exists = jnp.logical_and(r_w >= 1, r_w <= own)

    pltpu.sync_copy(ci_hbm.at[wid], civ)
    pltpu.sync_copy(cb_hbm.at[wid], cbv)
    pltpu.sync_copy(cc_hbm.at[wid], ccv)
    ncand = ccv[...][0]

    def scanbody(j, carry):
        val, cnt_before = carry
        cb = cbv[pl.ds(j * 16, 16)]
        ci = civ[pl.ds(j * 16, 16)]
        ordv = j * 16 + lane
        mask = jnp.logical_and(cb == t, ordv < ncand)
        rank = plsc.cumsum(jnp.where(mask, jnp.int32(1), jnp.int32(0)))
        hit = jnp.logical_and(mask, cnt_before + rank == r_w)
        val = val + jnp.sum(jnp.where(hit, ci, jnp.int32(0)))
        return val, cnt_before + rank[15]

    nit = lax.shift_right_logical(ncand + 15, 4)
    val, _ = lax.fori_loop(0, nit, scanbody, (jnp.int32(0), jnp.int32(0)))
    val = jnp.where(exists, val, jnp.int32(0))
    tout[...] = (jnp.where(lane == 0, t, jnp.int32(0))
                 + jnp.where(lane == 1, val, jnp.int32(0)))
    pltpu.sync_copy(tout, out_hbm.at[wid])


BLK_R = 128


def _finalize_body(t_ref, x_ref, o_ref):
    t = t_ref[0, 0]
    im = t_ref[0, 1]
    for w in range(1, NW):
        im = im + t_ref[w, 1]
    acts = jnp.maximum(x_ref[...], 0.0)
    bits = lax.bitcast_convert_type(acts, jnp.int32)
    ridx = lax.broadcasted_iota(jnp.int32, (BLK_R, COLS), 0)
    cidx = lax.broadcasted_iota(jnp.int32, (BLK_R, COLS), 1)
    flat = (pl.program_id(0) * BLK_R + ridx) * COLS + cidx
    keep = jnp.logical_or(
        bits > t, jnp.logical_and(bits == t, flat <= im)
    )
    o_ref[...] = jnp.where(keep, acts, 0.0)


def _finalize(tmeta, x):
    return pl.pallas_call(
        _finalize_body,
        grid=(ROWS // BLK_R,),
        in_specs=[
            pl.BlockSpec(memory_space=pltpu.SMEM),
            pl.BlockSpec((BLK_R, COLS), lambda i: (i, 0)),
        ],
        out_specs=pl.BlockSpec((BLK_R, COLS), lambda i: (i, 0)),
        out_shape=jax.ShapeDtypeStruct((ROWS, COLS), jnp.float32),
    )(tmeta, x)


def kernel(x):
    xf = x.reshape(-1)
    h1 = _hist1_kernel(xf).reshape(-1)
    h2 = _hist2_kernel(xf, h1).reshape(-1)
    h3, ci, cb, cc = _hist3_kernel(xf, h1, h2)
    tmeta = _thresh_kernel(h1, h2, h3.reshape(-1), ci, cb, cc)
    return _finalize(tmeta, x)
